# baseline (speedup 1.0000x reference)
import functools

import jax
import jax.numpy as jnp
from jax import lax
from jax.experimental import pallas as pl
from jax.experimental.pallas import tpu as pltpu
from jax.experimental.pallas import tpu_sc as plsc

N = 10000
E = 320000
D = 128
H = 256
NPAD = 10240
NW = 32
EPT = E // NW
RPT = -(-EPT // 128)
ZS = NPAD // 16
CH = 1024
NF = N // CH
TAIL = N - NF * CH


def _mask_body(src_hbm, dst_hbm, cv_hbm, out_hbm,
               src_v, cbuf, sbuf, zbuf, cv_v, shared):
    c = lax.axis_index("c")
    s = lax.axis_index("s")
    wid = s * 2 + c
    base = wid * EPT
    pltpu.sync_copy(src_hbm.at[pl.ds(base, EPT)], src_v)
    pltpu.sync_copy(cv_hbm, cv_v)
    cvv = cv_v[...]
    zeros = jnp.zeros((16,), jnp.float32)
    ones = jnp.ones((16,), jnp.float32)

    def zero_body(i, carry):
        zbuf[pl.ds(i * 16, 16)] = zeros
        return carry

    lax.fori_loop(0, ZS // 16, zero_body, 0)
    pltpu.sync_copy(zbuf, shared.at[pl.ds(s * ZS, ZS)])
    plsc.subcore_barrier()

    def row_body(j, carry):
        off = jnp.minimum(j * 128, EPT - 128)
        anyhit = src_v[pl.ds(off, 16)] == cvv
        for k in range(1, 8):
            anyhit = jnp.logical_or(anyhit, src_v[pl.ds(off + k * 16, 16)] == cvv)
        nhit = plsc.all_reduce_population_count(anyhit)

        @pl.when(nhit[0] > 0)
        def _():
            for k in range(8):
                sv = src_v[pl.ds(off + k * 16, 16)]
                cbuf[0, pl.ds(k * 16, 16)] = jnp.where(sv == cvv, ones, zeros)
            pltpu.sync_copy(dst_hbm.at[pl.ds(base + off, 128)], sbuf.at[0])
            pltpu.sync_copy(cbuf.at[0], shared.at[sbuf.at[0]], add=True)

        return carry

    lax.fori_loop(0, RPT, row_body, 0)
    plsc.subcore_barrier()

    @pl.when(s == 0)
    def _():
        pltpu.sync_copy(shared, out_hbm.at[c])


@functools.cache
def _mask_kernel():
    return pl.kernel(
        _mask_body,
        mesh=plsc.VectorSubcoreMesh(core_axis_name="c", subcore_axis_name="s"),
        compiler_params=pltpu.CompilerParams(needs_layout_passes=False),
        out_type=jax.ShapeDtypeStruct((2, NPAD), jnp.float32),
        scratch_types=[
            pltpu.VMEM((EPT,), jnp.int32),
            pltpu.VMEM((1, 128), jnp.float32),
            pltpu.VMEM((1, 128), jnp.int32),
            pltpu.VMEM((ZS,), jnp.float32),
            pltpu.VMEM((16,), jnp.int32),
            pltpu.VMEM_SHARED((NPAD,), jnp.float32),
        ],
    )


def _tc_body(v_ref, w1_ref, b1_ref, w2t_ref, b2_ref, part_ref, out_ref, scores):
    def chunk(base, rows):
        vch = v_ref[pl.ds(base, rows), :]
        h = lax.dot_general(
            w1_ref[...], vch, (((0,), (1,)), ((), ())),
            preferred_element_type=jnp.float32,
        )
        h = h + b1_ref[...]
        h = jnp.where(h > 0, h, 0.1 * h)
        s = lax.dot_general(
            w2t_ref[...], h, (((1,), (0,)), ((), ())),
            preferred_element_type=jnp.float32,
        )
        scores[0, pl.ds(base, rows)] = s[0, :] + b2_ref[0, 0]

    def chunk_body(ci, carry):
        chunk(ci * CH, CH)
        return carry

    lax.fori_loop(0, NF, chunk_body, 0)
    chunk(NF * CH, TAIL)

    nbr = jnp.sum(part_ref[...], axis=0, keepdims=True)
    idx = lax.broadcasted_iota(jnp.int32, (1, NPAD), 1)
    sc = scores[...]
    logits = jnp.where(idx < N, jnp.where(nbr > 0, sc, -1e9), -jnp.inf)
    m = jnp.max(logits)
    e = jnp.exp(logits - m)
    out_ref[...] = e / jnp.sum(e)


def _tc_call(v, w1, b1_col, w2t, b2, partials):
    return pl.pallas_call(
        _tc_body,
        out_shape=jax.ShapeDtypeStruct((1, NPAD), jnp.float32),
        scratch_shapes=[pltpu.VMEM((1, NPAD), jnp.float32)],
    )(v, w1, b1_col, w2t, b2, partials)


def kernel(vertices, edge_index, current_vertex,
           W1c, b1c, W2c, b2c, W3c, b3c, Wm1, bm1, Wm2, bm2):
    src = edge_index[0]
    dst = edge_index[1]
    cv_vec = jnp.full((16,), current_vertex, dtype=jnp.int32)
    partials = _mask_kernel()(src, dst, cv_vec)

    v = vertices.astype(jnp.float32)
    b1_col = bm1.reshape(H, 1)
    w2t = Wm2.reshape(1, H)
    b2 = bm2.reshape(1, 1)
    probs = _tc_call(v, Wm1, b1_col, w2t, b2, partials)
    return probs[0, :N]


# speedup vs baseline: 1.4394x; 1.4394x over previous
import functools

import jax
import jax.numpy as jnp
from jax import lax
from jax.experimental import pallas as pl
from jax.experimental.pallas import tpu as pltpu
from jax.experimental.pallas import tpu_sc as plsc

N = 10000
E = 320000
D = 128
H = 256
NPAD = 10240
NW = 32
EPT = E // NW
RPT = -(-EPT // 128)
ZS = NPAD // 16
CH = 1024
NF = N // CH
TAIL = N - NF * CH
EC = 32000
EG = E // EC


def _repack_body(ei_ref, src_ref, dst_ref):
    i = pl.program_id(0)
    x = ei_ref[...]
    src_ref[pl.ds(i * EC, EC)] = x[0]
    dst_ref[pl.ds(i * EC, EC)] = x[1]


def _repack_call(edge_index):
    return pl.pallas_call(
        _repack_body,
        grid=(EG,),
        in_specs=[pl.BlockSpec((2, EC), lambda i: (0, i))],
        out_specs=[
            pl.BlockSpec((E,), lambda i: (0,)),
            pl.BlockSpec((E,), lambda i: (0,)),
        ],
        out_shape=[
            jax.ShapeDtypeStruct((E,), jnp.int32),
            jax.ShapeDtypeStruct((E,), jnp.int32),
        ],
    )(edge_index)


def _mask_body(src_hbm, dst_hbm, cv_hbm, out_hbm,
               src_v, cbuf, sbuf, zbuf, cv_v, shared):
    c = lax.axis_index("c")
    s = lax.axis_index("s")
    wid = s * 2 + c
    base = wid * EPT
    pltpu.sync_copy(src_hbm.at[pl.ds(base, EPT)], src_v)
    pltpu.sync_copy(cv_hbm, cv_v)
    cvv = cv_v[...]
    zeros = jnp.zeros((16,), jnp.float32)
    ones = jnp.ones((16,), jnp.float32)

    def zero_body(i, carry):
        zbuf[pl.ds(i * 16, 16)] = zeros
        return carry

    lax.fori_loop(0, ZS // 16, zero_body, 0)
    pltpu.sync_copy(zbuf, shared.at[pl.ds(s * ZS, ZS)])
    plsc.subcore_barrier()

    def row_body(j, carry):
        off = jnp.minimum(j * 128, EPT - 128)
        anyhit = src_v[pl.ds(off, 16)] == cvv
        for k in range(1, 8):
            anyhit = jnp.logical_or(anyhit, src_v[pl.ds(off + k * 16, 16)] == cvv)
        nhit = plsc.all_reduce_population_count(anyhit)

        @pl.when(nhit[0] > 0)
        def _():
            for k in range(8):
                sv = src_v[pl.ds(off + k * 16, 16)]
                cbuf[0, pl.ds(k * 16, 16)] = jnp.where(sv == cvv, ones, zeros)
            pltpu.sync_copy(dst_hbm.at[pl.ds(base + off, 128)], sbuf.at[0])
            pltpu.sync_copy(cbuf.at[0], shared.at[sbuf.at[0]], add=True)

        return carry

    lax.fori_loop(0, RPT, row_body, 0)
    plsc.subcore_barrier()

    @pl.when(s == 0)
    def _():
        pltpu.sync_copy(shared, out_hbm.at[c])


@functools.cache
def _mask_kernel():
    return pl.kernel(
        _mask_body,
        mesh=plsc.VectorSubcoreMesh(core_axis_name="c", subcore_axis_name="s"),
        compiler_params=pltpu.CompilerParams(needs_layout_passes=False),
        out_type=jax.ShapeDtypeStruct((2, NPAD), jnp.float32),
        scratch_types=[
            pltpu.VMEM((EPT,), jnp.int32),
            pltpu.VMEM((1, 128), jnp.float32),
            pltpu.VMEM((1, 128), jnp.int32),
            pltpu.VMEM((ZS,), jnp.float32),
            pltpu.VMEM((16,), jnp.int32),
            pltpu.VMEM_SHARED((NPAD,), jnp.float32),
        ],
    )


def _scores_body(v_ref, w1_ref, b1_ref, w2t_ref, b2_ref, out_ref):
    def chunk(base, rows):
        vch = v_ref[pl.ds(base, rows), :]
        h = lax.dot_general(
            w1_ref[...], vch, (((0,), (1,)), ((), ())),
            preferred_element_type=jnp.float32,
        )
        h = h + b1_ref[...]
        h = jnp.where(h > 0, h, 0.1 * h)
        s = lax.dot_general(
            w2t_ref[...], h, (((1,), (0,)), ((), ())),
            preferred_element_type=jnp.float32,
        )
        out_ref[0, pl.ds(base, rows)] = s[0, :] + b2_ref[0, 0]

    def chunk_body(ci, carry):
        chunk(ci * CH, CH)
        return carry

    lax.fori_loop(0, NF, chunk_body, 0)
    chunk(NF * CH, TAIL)


def _scores_call(v, w1, b1_col, w2t, b2):
    return pl.pallas_call(
        _scores_body,
        out_shape=jax.ShapeDtypeStruct((1, NPAD), jnp.float32),
    )(v, w1, b1_col, w2t, b2)


def _softmax_body(sc_ref, part_ref, out_ref):
    nbr = jnp.sum(part_ref[...], axis=0, keepdims=True)
    idx = lax.broadcasted_iota(jnp.int32, (1, NPAD), 1)
    sc = sc_ref[...]
    logits = jnp.where(idx < N, jnp.where(nbr > 0, sc, -1e9), -jnp.inf)
    m = jnp.max(logits)
    e = jnp.exp(logits - m)
    p = e / jnp.sum(e)
    out_ref[...] = p[0, :N]


def _softmax_call(scores, partials):
    return pl.pallas_call(
        _softmax_body,
        out_shape=jax.ShapeDtypeStruct((N,), jnp.float32),
    )(scores, partials)


def kernel(vertices, edge_index, current_vertex,
           W1c, b1c, W2c, b2c, W3c, b3c, Wm1, bm1, Wm2, bm2):
    src, dst = _repack_call(edge_index)
    cv_vec = jnp.full((16,), current_vertex, dtype=jnp.int32)
    partials = _mask_kernel()(src, dst, cv_vec)

    v = vertices.astype(jnp.float32)
    b1_col = bm1.reshape(H, 1)
    w2t = Wm2.reshape(1, H)
    b2 = bm2.reshape(1, 1)
    scores = _scores_call(v, Wm1, b1_col, w2t, b2)
    return _softmax_call(scores, partials)


# speedup vs baseline: 1.4887x; 1.0342x over previous
import functools

import jax
import jax.numpy as jnp
from jax import lax
from jax.experimental import pallas as pl
from jax.experimental.pallas import tpu as pltpu
from jax.experimental.pallas import tpu_sc as plsc

N = 10000
E = 320000
D = 128
H = 256
NPAD = 10240
NW = 32
LSTEP = 9984
LANES = 10496
GRP = 512
NG = -(-LANES // GRP)
ZS = NPAD // 16
CH = 1024
NF = N // CH
TAIL = N - NF * CH


def _mask_body(ei_hbm, cv_hbm, out_hbm, ebuf, cbuf, sbuf, zbuf, cv_v, shared):
    c = lax.axis_index("c")
    s = lax.axis_index("s")
    wid = s * 2 + c
    base = wid * LSTEP
    pltpu.sync_copy(ei_hbm.at[:, pl.ds(base, LANES)], ebuf)
    pltpu.sync_copy(cv_hbm, cv_v)
    cvv = cv_v[...]
    zeros = jnp.zeros((16,), jnp.float32)
    ones = jnp.ones((16,), jnp.float32)

    def zero_body(i, carry):
        zbuf[pl.ds(i * 16, 16)] = zeros
        return carry

    lax.fori_loop(0, ZS // 16, zero_body, 0)
    pltpu.sync_copy(zbuf, shared.at[pl.ds(s * ZS, ZS)])
    plsc.subcore_barrier()

    def grp_body(g, carry):
        off = jnp.minimum(g * GRP, LANES - GRP)
        anyhit = ebuf[0, pl.ds(off, 16)] == cvv
        for k in range(1, GRP // 16):
            anyhit = jnp.logical_or(anyhit, ebuf[0, pl.ds(off + k * 16, 16)] == cvv)
        nhit = plsc.all_reduce_population_count(anyhit)

        @pl.when(nhit[0] > 0)
        def _():
            for r in range(GRP // 128):
                for k in range(8):
                    sv = ebuf[0, pl.ds(off + r * 128 + k * 16, 16)]
                    cbuf[r, pl.ds(k * 16, 16)] = jnp.where(sv == cvv, ones, zeros)
                pltpu.sync_copy(ei_hbm.at[:, pl.ds(base + off + r * 128, 128)], sbuf)
                pltpu.sync_copy(cbuf.at[r], shared.at[sbuf.at[1]], add=True)

        return carry

    lax.fori_loop(0, NG, grp_body, 0)
    plsc.subcore_barrier()

    @pl.when(s == 0)
    def _():
        pltpu.sync_copy(shared, out_hbm.at[c])


@functools.cache
def _mask_kernel():
    return pl.kernel(
        _mask_body,
        mesh=plsc.VectorSubcoreMesh(core_axis_name="c", subcore_axis_name="s"),
        compiler_params=pltpu.CompilerParams(needs_layout_passes=False),
        out_type=jax.ShapeDtypeStruct((2, NPAD), jnp.float32),
        scratch_types=[
            pltpu.VMEM((2, LANES), jnp.int32),
            pltpu.VMEM((GRP // 128, 128), jnp.float32),
            pltpu.VMEM((2, 128), jnp.int32),
            pltpu.VMEM((ZS,), jnp.float32),
            pltpu.VMEM((16,), jnp.int32),
            pltpu.VMEM_SHARED((NPAD,), jnp.float32),
        ],
    )


def _scores_body(v_ref, w1_ref, b1_ref, w2t_ref, b2_ref, out_ref):
    def chunk(base, rows):
        vch = v_ref[pl.ds(base, rows), :]
        h = lax.dot_general(
            w1_ref[...], vch, (((0,), (1,)), ((), ())),
            preferred_element_type=jnp.float32,
        )
        h = h + b1_ref[...]
        h = jnp.where(h > 0, h, 0.1 * h)
        s = lax.dot_general(
            w2t_ref[...], h, (((1,), (0,)), ((), ())),
            preferred_element_type=jnp.float32,
        )
        out_ref[0, pl.ds(base, rows)] = s[0, :] + b2_ref[0, 0]

    def chunk_body(ci, carry):
        chunk(ci * CH, CH)
        return carry

    lax.fori_loop(0, NF, chunk_body, 0)
    chunk(NF * CH, TAIL)


def _scores_call(v, w1, b1_col, w2t, b2):
    return pl.pallas_call(
        _scores_body,
        out_shape=jax.ShapeDtypeStruct((1, NPAD), jnp.float32),
    )(v, w1, b1_col, w2t, b2)


def _softmax_body(sc_ref, part_ref, out_ref):
    nbr = jnp.sum(part_ref[...], axis=0, keepdims=True)
    idx = lax.broadcasted_iota(jnp.int32, (1, NPAD), 1)
    sc = sc_ref[...]
    logits = jnp.where(idx < N, jnp.where(nbr > 0, sc, -1e9), -jnp.inf)
    m = jnp.max(logits)
    e = jnp.exp(logits - m)
    p = e / jnp.sum(e)
    out_ref[...] = p[0, :N]


def _softmax_call(scores, partials):
    return pl.pallas_call(
        _softmax_body,
        out_shape=jax.ShapeDtypeStruct((N,), jnp.float32),
    )(scores, partials)


def kernel(vertices, edge_index, current_vertex,
           W1c, b1c, W2c, b2c, W3c, b3c, Wm1, bm1, Wm2, bm2):
    cv_vec = jnp.full((16,), current_vertex, dtype=jnp.int32)
    partials = _mask_kernel()(edge_index, cv_vec)

    v = vertices.astype(jnp.float32)
    b1_col = bm1.reshape(H, 1)
    w2t = Wm2.reshape(1, H)
    b2 = bm2.reshape(1, 1)
    scores = _scores_call(v, Wm1, b1_col, w2t, b2)
    return _softmax_call(scores, partials)
